# baseline (speedup 1.0000x reference)
import functools
import math

import jax
import jax.numpy as jnp
from jax import lax
from jax.experimental import pallas as pl
from jax.experimental.pallas import tpu as pltpu
from jax.experimental.pallas import tpu_sc as plsc

D_MODEL = 64
NUM_CORES = 2
NUM_SUBCORES = 16
NUM_WORKERS = NUM_CORES * NUM_SUBCORES
SEQ = 200
SPLIT = 128
LANES = 16
NBUF = 4
LEAD = 2


def _emb_body(tok_hbm, table_hbm, out_hbm, idx_v, rows_v, *sems):
    scale = math.sqrt(D_MODEL)
    gsems, osems = sems[:NBUF], sems[NBUF:]
    rows_per_w = tok_hbm.shape[0] // NUM_WORKERS
    wid = lax.axis_index("s") * NUM_CORES + lax.axis_index("c")
    row0 = wid * rows_per_w
    pltpu.sync_copy(tok_hbm.at[pl.ds(row0, rows_per_w)], idx_v)

    def gather_pair(r, t, start):
        a = pltpu.make_async_copy(
            table_hbm.at[idx_v.at[r, pl.ds(0, SPLIT)]],
            rows_v.at[t, pl.ds(0, SPLIT)], gsems[t])
        b = pltpu.make_async_copy(
            table_hbm.at[idx_v.at[r, pl.ds(SPLIT, SEQ - SPLIT)]],
            rows_v.at[t, pl.ds(SPLIT, SEQ - SPLIT)], gsems[t])
        if start:
            a.start()
            b.start()
        else:
            a.wait()
            b.wait()

    for t in range(LEAD):
        gather_pair(t, t, True)

    @pl.loop(0, rows_per_w, step=NBUF)
    def group(r0):
        for t in range(NBUF):
            r = r0 + t
            gather_pair(r, t, False)

            @pl.loop(0, SEQ, unroll=8)
            def row(q):
                for c in range(D_MODEL // LANES):
                    sl = pl.ds(c * LANES, LANES)
                    rows_v[t, q, sl] = rows_v[t, q, sl] * scale

            pltpu.async_copy(rows_v.at[t], out_hbm.at[row0 + r], osems[t])

            rn = r + LEAD
            tn = (t + LEAD) % NBUF

            @pl.when(rn < rows_per_w)
            def _():
                @pl.when(rn >= NBUF)
                def _():
                    pltpu.make_async_copy(
                        rows_v.at[tn], out_hbm.at[row0 + rn - NBUF],
                        osems[tn]).wait()
                gather_pair(rn, tn, True)

    for t in range(NBUF):
        pltpu.make_async_copy(
            rows_v.at[t], out_hbm.at[row0 + rows_per_w - NBUF + t],
            osems[t]).wait()


def kernel(tokens, embedding_weight):
    b, s = tokens.shape
    tok = tokens.astype(jnp.int32)

    mesh = plsc.VectorSubcoreMesh(
        core_axis_name="c", subcore_axis_name="s",
        num_cores=NUM_CORES, num_subcores=NUM_SUBCORES)

    emb = functools.partial(
        pl.kernel,
        out_type=jax.ShapeDtypeStruct((b, s, D_MODEL), jnp.float32),
        mesh=mesh,
        scratch_types=[
            pltpu.VMEM((b // NUM_WORKERS, s), jnp.int32),
            pltpu.VMEM((NBUF, SEQ, D_MODEL), jnp.float32),
        ] + [pltpu.SemaphoreType.DMA] * (2 * NBUF),
        compiler_params=pltpu.CompilerParams(use_tc_tiling_on_sc=False),
    )(_emb_body)

    return emb(tok, embedding_weight)
